# baseline (speedup 1.0000x reference)
import functools

import jax
import jax.numpy as jnp
from jax import lax
from jax.experimental import pallas as pl
from jax.experimental.pallas import tpu as pltpu
from jax.experimental.pallas import tpu_sc as plsc

N_NODES = 10000
DIM = 128
N_LAYERS = 3

NC = 2
NS = 16
NW = NC * NS
CHUNK = 128

ACC_ROWS = 10240
DUMMY_ROW = N_NODES
ROWS_PER_TILE = ACC_ROWS // NS
CNT_W = DIM
GROUP = 8

C0 = 112
C1 = 48
TOT_CHUNKS = NS * (C0 + C1)


def _sc_layer_body(h_hbm, src_hbm, dst_hbm, part_out,
                   src_v, dst_v, rows_v, acc_sh, sem0, sem1):
    cid = lax.axis_index("c")
    sid = lax.axis_index("s")
    start = lax.select(cid == 0, sid * C0, NS * C0 + sid * C1)
    n_groups = lax.select(cid == 0, C0 // GROUP, C1 // GROUP)
    sems = (sem0, sem1)

    def _zero_fill(i, carry):
        for j in range(DIM // 16):
            rows_v[0, i, pl.ds(j * 16, 16)] = jnp.zeros((16,), jnp.float32)
        return carry

    lax.fori_loop(0, CHUNK, _zero_fill, 0)
    for k in range(ROWS_PER_TILE // CHUNK):
        pltpu.sync_copy(rows_v.at[0],
                        acc_sh.at[pl.ds(sid * ROWS_PER_TILE + k * CHUNK, CHUNK)])

    plsc.subcore_barrier()

    pltpu.sync_copy(src_hbm.at[pl.ds(start, GROUP)], src_v.at[0])
    pltpu.sync_copy(dst_hbm.at[pl.ds(start, GROUP)], dst_v.at[0])
    pltpu.async_copy(h_hbm.at[src_v.at[0, 0]], rows_v.at[0], sem0)

    def _edge_group(g, carry):
        slot = lax.rem(g, 2)
        nslot = lax.rem(g + 1, 2)
        not_last = g + 1 < n_groups

        @pl.when(not_last)
        def _():
            pltpu.sync_copy(src_hbm.at[pl.ds(start + (g + 1) * GROUP, GROUP)],
                            src_v.at[nslot])
            pltpu.sync_copy(dst_hbm.at[pl.ds(start + (g + 1) * GROUP, GROUP)],
                            dst_v.at[nslot])

        for b in range(GROUP):
            cur = rows_v.at[b % 2]
            nxt = rows_v.at[(b + 1) % 2]
            if b + 1 < GROUP:
                pltpu.async_copy(h_hbm.at[src_v.at[slot, b + 1]], nxt,
                                 sems[(b + 1) % 2])
            else:
                @pl.when(not_last)
                def _():
                    pltpu.async_copy(h_hbm.at[src_v.at[nslot, 0]], nxt,
                                     sems[(b + 1) % 2])
            pltpu.make_async_copy(h_hbm.at[src_v.at[slot, b]], cur,
                                  sems[b % 2]).wait()
            pltpu.sync_copy(cur, acc_sh.at[dst_v.at[slot, b]], add=True)
        return carry

    lax.fori_loop(0, n_groups, _edge_group, 0)

    plsc.subcore_barrier()

    base = sid * ROWS_PER_TILE
    pltpu.sync_copy(acc_sh.at[pl.ds(base, ROWS_PER_TILE)],
                    part_out.at[cid, pl.ds(base, ROWS_PER_TILE)])


def _make_sc_layer():
    mesh = plsc.VectorSubcoreMesh(core_axis_name="c", subcore_axis_name="s")
    return pl.kernel(
        _sc_layer_body,
        out_type=(jax.ShapeDtypeStruct((NC, ACC_ROWS, DIM), jnp.float32),),
        mesh=mesh,
        scratch_types=(
            pltpu.VMEM((2, GROUP, CHUNK), jnp.int32),
            pltpu.VMEM((2, GROUP, CHUNK), jnp.int32),
            pltpu.VMEM((2, CHUNK, DIM), jnp.float32),
            pltpu.VMEM_SHARED((ACC_ROWS, DIM), jnp.float32),
            pltpu.SemaphoreType.DMA,
            pltpu.SemaphoreType.DMA,
        ),
    )


def _sc_counts_body(dst_hbm, cnt_out, dst_v, rows_v, cnt_sh):
    cid = lax.axis_index("c")
    sid = lax.axis_index("s")
    wid = cid * NS + sid
    per_worker = dst_hbm.shape[0] // NW
    start = wid * per_worker
    n_groups = per_worker // GROUP

    def _fill(val):
        def _body(i, carry):
            for j in range(DIM // 16):
                rows_v[i, pl.ds(j * 16, 16)] = jnp.full((16,), val, jnp.float32)
            return carry
        lax.fori_loop(0, CHUNK, _body, 0)

    _fill(0.0)
    for k in range(ROWS_PER_TILE // CHUNK):
        pltpu.sync_copy(rows_v, cnt_sh.at[pl.ds(sid * ROWS_PER_TILE + k * CHUNK, CHUNK)])
    _fill(1.0)

    plsc.subcore_barrier()

    def _edge_group(g, carry):
        pltpu.sync_copy(dst_hbm.at[pl.ds(start + g * GROUP, GROUP)], dst_v)
        for b in range(GROUP):
            pltpu.sync_copy(rows_v, cnt_sh.at[dst_v.at[b]], add=True)
        return carry

    lax.fori_loop(0, n_groups, _edge_group, 0)

    plsc.subcore_barrier()

    base = sid * ROWS_PER_TILE
    pltpu.sync_copy(cnt_sh.at[pl.ds(base, ROWS_PER_TILE)],
                    cnt_out.at[cid, pl.ds(base, ROWS_PER_TILE)])


def _make_sc_counts():
    mesh = plsc.VectorSubcoreMesh(core_axis_name="c", subcore_axis_name="s")
    return pl.kernel(
        _sc_counts_body,
        out_type=(jax.ShapeDtypeStruct((NC, ACC_ROWS, CNT_W), jnp.float32),),
        mesh=mesh,
        scratch_types=(
            pltpu.VMEM((GROUP, CHUNK), jnp.int32),
            pltpu.VMEM((CHUNK, CNT_W), jnp.float32),
            pltpu.VMEM_SHARED((ACC_ROWS, CNT_W), jnp.float32),
        ),
    )


def _tc_combine_body(first, part_ref, cnt_ref, fin_ref, h_ref, fout_ref):
    p = part_ref[0] + part_ref[1]
    c = cnt_ref[0, :, 0:1] + cnt_ref[1, :, 0:1]
    h = p / jnp.maximum(c, 1.0)
    h_ref[...] = h
    if first:
        fout_ref[...] = (fin_ref[...] + h) * 0.25
    else:
        fout_ref[...] = fin_ref[...] + h * 0.25


def _make_tc_combine(first):
    BR = 1000
    grid = (N_NODES // BR,)
    return pl.pallas_call(
        functools.partial(_tc_combine_body, first),
        grid=grid,
        in_specs=[
            pl.BlockSpec((NC, BR, DIM), lambda i: (0, i, 0)),
            pl.BlockSpec((NC, BR, CNT_W), lambda i: (0, i, 0)),
            pl.BlockSpec((BR, DIM), lambda i: (i, 0)),
        ],
        out_specs=[
            pl.BlockSpec((BR, DIM), lambda i: (i, 0)),
            pl.BlockSpec((BR, DIM), lambda i: (i, 0)),
        ],
        out_shape=[
            jax.ShapeDtypeStruct((N_NODES, DIM), jnp.float32),
            jax.ShapeDtypeStruct((N_NODES, DIM), jnp.float32),
        ],
    )


def kernel(x, edge_index):
    n_edges = edge_index.shape[1]
    pad_e = TOT_CHUNKS * CHUNK
    n_pad = pad_e - n_edges

    src = edge_index[0].astype(jnp.int32)
    dst = edge_index[1].astype(jnp.int32)
    src = jnp.concatenate(
        [src, jnp.zeros((n_pad,), jnp.int32)]).reshape(TOT_CHUNKS, CHUNK)
    pad_dst = DUMMY_ROW + (jnp.arange(n_pad, dtype=jnp.int32) % (ACC_ROWS - N_NODES))
    dst = jnp.concatenate([dst, pad_dst]).reshape(TOT_CHUNKS, CHUNK)

    sc_layer = _make_sc_layer()
    sc_counts = _make_sc_counts()
    tc_first = _make_tc_combine(first=True)
    tc_rest = _make_tc_combine(first=False)

    (cnt,) = sc_counts(dst)
    (part,) = sc_layer(x, src, dst)
    h, fin = tc_first(part, cnt, x)
    for _ in range(N_LAYERS - 1):
        (part,) = sc_layer(h, src, dst)
        h, fin = tc_rest(part, cnt, fin)
    return fin


# speedup vs baseline: 1.1485x; 1.1485x over previous
import functools

import jax
import jax.numpy as jnp
from jax import lax
from jax.experimental import pallas as pl
from jax.experimental.pallas import tpu as pltpu
from jax.experimental.pallas import tpu_sc as plsc

N_NODES = 10000
DIM = 128
N_LAYERS = 3

NC = 2
NS = 16
NW = NC * NS
CHUNK = 128

ACC_ROWS = 10240
DUMMY_ROW = N_NODES
ROWS_PER_TILE = ACC_ROWS // NS
CNT_W = DIM
GROUP = 8

C0 = 144
C1 = 16
TOT_CHUNKS = NS * (C0 + C1)


def _sc_layer_body(h_hbm, src_hbm, dst_hbm, part_out,
                   src_v, dst_v, rows_v, acc_sh, sem0, sem1):
    cid = lax.axis_index("c")
    sid = lax.axis_index("s")
    start = lax.select(cid == 0, sid * C0, NS * C0 + sid * C1)
    n_groups = lax.select(cid == 0, C0 // GROUP, C1 // GROUP)
    sems = (sem0, sem1)

    def _zero_fill(i, carry):
        for j in range(DIM // 16):
            rows_v[0, i, pl.ds(j * 16, 16)] = jnp.zeros((16,), jnp.float32)
        return carry

    lax.fori_loop(0, CHUNK, _zero_fill, 0)
    for k in range(ROWS_PER_TILE // CHUNK):
        pltpu.sync_copy(rows_v.at[0],
                        acc_sh.at[pl.ds(sid * ROWS_PER_TILE + k * CHUNK, CHUNK)])

    plsc.subcore_barrier()

    pltpu.sync_copy(src_hbm.at[pl.ds(start, GROUP)], src_v.at[0])
    pltpu.sync_copy(dst_hbm.at[pl.ds(start, GROUP)], dst_v.at[0])
    pltpu.async_copy(h_hbm.at[src_v.at[0, 0]], rows_v.at[0], sem0)

    def _edge_group(g, carry):
        slot = lax.rem(g, 2)
        nslot = lax.rem(g + 1, 2)
        not_last = g + 1 < n_groups

        @pl.when(not_last)
        def _():
            pltpu.sync_copy(src_hbm.at[pl.ds(start + (g + 1) * GROUP, GROUP)],
                            src_v.at[nslot])
            pltpu.sync_copy(dst_hbm.at[pl.ds(start + (g + 1) * GROUP, GROUP)],
                            dst_v.at[nslot])

        for b in range(GROUP):
            cur = rows_v.at[b % 2]
            nxt = rows_v.at[(b + 1) % 2]
            if b + 1 < GROUP:
                pltpu.async_copy(h_hbm.at[src_v.at[slot, b + 1]], nxt,
                                 sems[(b + 1) % 2])
            else:
                @pl.when(not_last)
                def _():
                    pltpu.async_copy(h_hbm.at[src_v.at[nslot, 0]], nxt,
                                     sems[(b + 1) % 2])
            pltpu.make_async_copy(h_hbm.at[src_v.at[slot, b]], cur,
                                  sems[b % 2]).wait()
            pltpu.sync_copy(cur, acc_sh.at[dst_v.at[slot, b]], add=True)
        return carry

    lax.fori_loop(0, n_groups, _edge_group, 0)

    plsc.subcore_barrier()

    base = sid * ROWS_PER_TILE
    pltpu.sync_copy(acc_sh.at[pl.ds(base, ROWS_PER_TILE)],
                    part_out.at[cid, pl.ds(base, ROWS_PER_TILE)])


def _make_sc_layer():
    mesh = plsc.VectorSubcoreMesh(core_axis_name="c", subcore_axis_name="s")
    return pl.kernel(
        _sc_layer_body,
        out_type=(jax.ShapeDtypeStruct((NC, ACC_ROWS, DIM), jnp.float32),),
        mesh=mesh,
        scratch_types=(
            pltpu.VMEM((2, GROUP, CHUNK), jnp.int32),
            pltpu.VMEM((2, GROUP, CHUNK), jnp.int32),
            pltpu.VMEM((2, CHUNK, DIM), jnp.float32),
            pltpu.VMEM_SHARED((ACC_ROWS, DIM), jnp.float32),
            pltpu.SemaphoreType.DMA,
            pltpu.SemaphoreType.DMA,
        ),
    )


def _sc_counts_body(dst_hbm, cnt_out, dst_v, rows_v, cnt_sh):
    cid = lax.axis_index("c")
    sid = lax.axis_index("s")
    wid = cid * NS + sid
    per_worker = dst_hbm.shape[0] // NW
    start = wid * per_worker
    n_groups = per_worker // GROUP

    def _fill(val):
        def _body(i, carry):
            for j in range(DIM // 16):
                rows_v[i, pl.ds(j * 16, 16)] = jnp.full((16,), val, jnp.float32)
            return carry
        lax.fori_loop(0, CHUNK, _body, 0)

    _fill(0.0)
    for k in range(ROWS_PER_TILE // CHUNK):
        pltpu.sync_copy(rows_v, cnt_sh.at[pl.ds(sid * ROWS_PER_TILE + k * CHUNK, CHUNK)])
    _fill(1.0)

    plsc.subcore_barrier()

    def _edge_group(g, carry):
        pltpu.sync_copy(dst_hbm.at[pl.ds(start + g * GROUP, GROUP)], dst_v)
        for b in range(GROUP):
            pltpu.sync_copy(rows_v, cnt_sh.at[dst_v.at[b]], add=True)
        return carry

    lax.fori_loop(0, n_groups, _edge_group, 0)

    plsc.subcore_barrier()

    base = sid * ROWS_PER_TILE
    pltpu.sync_copy(cnt_sh.at[pl.ds(base, ROWS_PER_TILE)],
                    cnt_out.at[cid, pl.ds(base, ROWS_PER_TILE)])


def _make_sc_counts():
    mesh = plsc.VectorSubcoreMesh(core_axis_name="c", subcore_axis_name="s")
    return pl.kernel(
        _sc_counts_body,
        out_type=(jax.ShapeDtypeStruct((NC, ACC_ROWS, CNT_W), jnp.float32),),
        mesh=mesh,
        scratch_types=(
            pltpu.VMEM((GROUP, CHUNK), jnp.int32),
            pltpu.VMEM((CHUNK, CNT_W), jnp.float32),
            pltpu.VMEM_SHARED((ACC_ROWS, CNT_W), jnp.float32),
        ),
    )


def _tc_combine_body(first, part_ref, cnt_ref, fin_ref, h_ref, fout_ref):
    p = part_ref[0] + part_ref[1]
    c = cnt_ref[0, :, 0:1] + cnt_ref[1, :, 0:1]
    h = p / jnp.maximum(c, 1.0)
    h_ref[...] = h
    if first:
        fout_ref[...] = (fin_ref[...] + h) * 0.25
    else:
        fout_ref[...] = fin_ref[...] + h * 0.25


def _make_tc_combine(first):
    BR = 1000
    grid = (N_NODES // BR,)
    return pl.pallas_call(
        functools.partial(_tc_combine_body, first),
        grid=grid,
        in_specs=[
            pl.BlockSpec((NC, BR, DIM), lambda i: (0, i, 0)),
            pl.BlockSpec((NC, BR, CNT_W), lambda i: (0, i, 0)),
            pl.BlockSpec((BR, DIM), lambda i: (i, 0)),
        ],
        out_specs=[
            pl.BlockSpec((BR, DIM), lambda i: (i, 0)),
            pl.BlockSpec((BR, DIM), lambda i: (i, 0)),
        ],
        out_shape=[
            jax.ShapeDtypeStruct((N_NODES, DIM), jnp.float32),
            jax.ShapeDtypeStruct((N_NODES, DIM), jnp.float32),
        ],
    )


def kernel(x, edge_index):
    n_edges = edge_index.shape[1]
    pad_e = TOT_CHUNKS * CHUNK
    n_pad = pad_e - n_edges

    src = edge_index[0].astype(jnp.int32)
    dst = edge_index[1].astype(jnp.int32)
    src = jnp.concatenate(
        [src, jnp.zeros((n_pad,), jnp.int32)]).reshape(TOT_CHUNKS, CHUNK)
    pad_dst = DUMMY_ROW + (jnp.arange(n_pad, dtype=jnp.int32) % (ACC_ROWS - N_NODES))
    dst = jnp.concatenate([dst, pad_dst]).reshape(TOT_CHUNKS, CHUNK)

    sc_layer = _make_sc_layer()
    sc_counts = _make_sc_counts()
    tc_first = _make_tc_combine(first=True)
    tc_rest = _make_tc_combine(first=False)

    (cnt,) = sc_counts(dst)
    (part,) = sc_layer(x, src, dst)
    h, fin = tc_first(part, cnt, x)
    for _ in range(N_LAYERS - 1):
        (part,) = sc_layer(h, src, dst)
        h, fin = tc_rest(part, cnt, fin)
    return fin


# speedup vs baseline: 1.1612x; 1.0110x over previous
import functools

import jax
import jax.numpy as jnp
from jax import lax
from jax.experimental import pallas as pl
from jax.experimental.pallas import tpu as pltpu
from jax.experimental.pallas import tpu_sc as plsc

N_NODES = 10000
DIM = 128
N_LAYERS = 3

NC = 2
NS = 16
NW = NC * NS
CHUNK = 128

ACC_ROWS = 10240
DUMMY_ROW = N_NODES
ROWS_PER_TILE = ACC_ROWS // NS
CNT_W = DIM
GROUP = 8

C0 = 152
C1 = 8
TOT_CHUNKS = NS * (C0 + C1)


def _sc_layer_body(h_hbm, src_hbm, dst_hbm, part_out,
                   src_v, dst_v, rows_v, acc_sh, sem0, sem1):
    cid = lax.axis_index("c")
    sid = lax.axis_index("s")
    start = lax.select(cid == 0, sid * C0, NS * C0 + sid * C1)
    n_groups = lax.select(cid == 0, C0 // GROUP, C1 // GROUP)
    sems = (sem0, sem1)

    def _zero_fill(i, carry):
        for j in range(DIM // 16):
            rows_v[0, i, pl.ds(j * 16, 16)] = jnp.zeros((16,), jnp.float32)
        return carry

    lax.fori_loop(0, CHUNK, _zero_fill, 0)
    for k in range(ROWS_PER_TILE // CHUNK):
        pltpu.sync_copy(rows_v.at[0],
                        acc_sh.at[pl.ds(sid * ROWS_PER_TILE + k * CHUNK, CHUNK)])

    plsc.subcore_barrier()

    pltpu.sync_copy(src_hbm.at[pl.ds(start, GROUP)], src_v.at[0])
    pltpu.sync_copy(dst_hbm.at[pl.ds(start, GROUP)], dst_v.at[0])
    pltpu.async_copy(h_hbm.at[src_v.at[0, 0]], rows_v.at[0], sem0)

    def _edge_group(g, carry):
        slot = lax.rem(g, 2)
        nslot = lax.rem(g + 1, 2)
        not_last = g + 1 < n_groups

        @pl.when(not_last)
        def _():
            pltpu.sync_copy(src_hbm.at[pl.ds(start + (g + 1) * GROUP, GROUP)],
                            src_v.at[nslot])
            pltpu.sync_copy(dst_hbm.at[pl.ds(start + (g + 1) * GROUP, GROUP)],
                            dst_v.at[nslot])

        for b in range(GROUP):
            cur = rows_v.at[b % 2]
            nxt = rows_v.at[(b + 1) % 2]
            if b + 1 < GROUP:
                pltpu.async_copy(h_hbm.at[src_v.at[slot, b + 1]], nxt,
                                 sems[(b + 1) % 2])
            else:
                @pl.when(not_last)
                def _():
                    pltpu.async_copy(h_hbm.at[src_v.at[nslot, 0]], nxt,
                                     sems[(b + 1) % 2])
            pltpu.make_async_copy(h_hbm.at[src_v.at[slot, b]], cur,
                                  sems[b % 2]).wait()
            pltpu.sync_copy(cur, acc_sh.at[dst_v.at[slot, b]], add=True)
        return carry

    lax.fori_loop(0, n_groups, _edge_group, 0)

    plsc.subcore_barrier()

    base = sid * ROWS_PER_TILE
    pltpu.sync_copy(acc_sh.at[pl.ds(base, ROWS_PER_TILE)],
                    part_out.at[cid, pl.ds(base, ROWS_PER_TILE)])


def _make_sc_layer():
    mesh = plsc.VectorSubcoreMesh(core_axis_name="c", subcore_axis_name="s")
    return pl.kernel(
        _sc_layer_body,
        out_type=(jax.ShapeDtypeStruct((NC, ACC_ROWS, DIM), jnp.float32),),
        mesh=mesh,
        scratch_types=(
            pltpu.VMEM((2, GROUP, CHUNK), jnp.int32),
            pltpu.VMEM((2, GROUP, CHUNK), jnp.int32),
            pltpu.VMEM((2, CHUNK, DIM), jnp.float32),
            pltpu.VMEM_SHARED((ACC_ROWS, DIM), jnp.float32),
            pltpu.SemaphoreType.DMA,
            pltpu.SemaphoreType.DMA,
        ),
    )


def _sc_counts_body(dst_hbm, cnt_out, dst_v, rows_v, cnt_sh):
    cid = lax.axis_index("c")
    sid = lax.axis_index("s")
    wid = cid * NS + sid
    per_worker = dst_hbm.shape[0] // NW
    start = wid * per_worker
    n_groups = per_worker // GROUP

    def _fill(val):
        def _body(i, carry):
            for j in range(DIM // 16):
                rows_v[i, pl.ds(j * 16, 16)] = jnp.full((16,), val, jnp.float32)
            return carry
        lax.fori_loop(0, CHUNK, _body, 0)

    _fill(0.0)
    for k in range(ROWS_PER_TILE // CHUNK):
        pltpu.sync_copy(rows_v, cnt_sh.at[pl.ds(sid * ROWS_PER_TILE + k * CHUNK, CHUNK)])
    _fill(1.0)

    plsc.subcore_barrier()

    def _edge_group(g, carry):
        pltpu.sync_copy(dst_hbm.at[pl.ds(start + g * GROUP, GROUP)], dst_v)
        for b in range(GROUP):
            pltpu.sync_copy(rows_v, cnt_sh.at[dst_v.at[b]], add=True)
        return carry

    lax.fori_loop(0, n_groups, _edge_group, 0)

    plsc.subcore_barrier()

    base = sid * ROWS_PER_TILE
    pltpu.sync_copy(cnt_sh.at[pl.ds(base, ROWS_PER_TILE)],
                    cnt_out.at[cid, pl.ds(base, ROWS_PER_TILE)])


def _make_sc_counts():
    mesh = plsc.VectorSubcoreMesh(core_axis_name="c", subcore_axis_name="s")
    return pl.kernel(
        _sc_counts_body,
        out_type=(jax.ShapeDtypeStruct((NC, ACC_ROWS, CNT_W), jnp.float32),),
        mesh=mesh,
        scratch_types=(
            pltpu.VMEM((GROUP, CHUNK), jnp.int32),
            pltpu.VMEM((CHUNK, CNT_W), jnp.float32),
            pltpu.VMEM_SHARED((ACC_ROWS, CNT_W), jnp.float32),
        ),
    )


def _tc_combine_body(first, part_ref, cnt_ref, fin_ref, h_ref, fout_ref):
    p = part_ref[0] + part_ref[1]
    c = cnt_ref[0, :, 0:1] + cnt_ref[1, :, 0:1]
    h = p / jnp.maximum(c, 1.0)
    h_ref[...] = h
    if first:
        fout_ref[...] = (fin_ref[...] + h) * 0.25
    else:
        fout_ref[...] = fin_ref[...] + h * 0.25


def _make_tc_combine(first):
    BR = 1000
    grid = (N_NODES // BR,)
    return pl.pallas_call(
        functools.partial(_tc_combine_body, first),
        grid=grid,
        in_specs=[
            pl.BlockSpec((NC, BR, DIM), lambda i: (0, i, 0)),
            pl.BlockSpec((NC, BR, CNT_W), lambda i: (0, i, 0)),
            pl.BlockSpec((BR, DIM), lambda i: (i, 0)),
        ],
        out_specs=[
            pl.BlockSpec((BR, DIM), lambda i: (i, 0)),
            pl.BlockSpec((BR, DIM), lambda i: (i, 0)),
        ],
        out_shape=[
            jax.ShapeDtypeStruct((N_NODES, DIM), jnp.float32),
            jax.ShapeDtypeStruct((N_NODES, DIM), jnp.float32),
        ],
    )


def kernel(x, edge_index):
    n_edges = edge_index.shape[1]
    pad_e = TOT_CHUNKS * CHUNK
    n_pad = pad_e - n_edges

    src = edge_index[0].astype(jnp.int32)
    dst = edge_index[1].astype(jnp.int32)
    src = jnp.concatenate(
        [src, jnp.zeros((n_pad,), jnp.int32)]).reshape(TOT_CHUNKS, CHUNK)
    pad_dst = DUMMY_ROW + (jnp.arange(n_pad, dtype=jnp.int32) % (ACC_ROWS - N_NODES))
    dst = jnp.concatenate([dst, pad_dst]).reshape(TOT_CHUNKS, CHUNK)

    sc_layer = _make_sc_layer()
    sc_counts = _make_sc_counts()
    tc_first = _make_tc_combine(first=True)
    tc_rest = _make_tc_combine(first=False)

    (cnt,) = sc_counts(dst)
    (part,) = sc_layer(x, src, dst)
    h, fin = tc_first(part, cnt, x)
    for _ in range(N_LAYERS - 1):
        (part,) = sc_layer(h, src, dst)
        h, fin = tc_rest(part, cnt, fin)
    return fin


# speedup vs baseline: 1.1617x; 1.0005x over previous
import functools

import jax
import jax.numpy as jnp
from jax import lax
from jax.experimental import pallas as pl
from jax.experimental.pallas import tpu as pltpu
from jax.experimental.pallas import tpu_sc as plsc

N_NODES = 10000
DIM = 128
N_LAYERS = 3

NC = 2
NS = 16
NW = NC * NS
CHUNK = 128

ACC_ROWS = 10240
DUMMY_ROW = N_NODES
ROWS_PER_TILE = ACC_ROWS // NS
CNT_W = DIM
GROUP = 8

C0 = 152
C1 = 8
TOT_CHUNKS = NS * (C0 + C1)


def _sc_layer_body(h_hbm, src_hbm, dst_hbm, part_out,
                   src_v, dst_v, rows_v, acc_sh, sem0, sem1, isem):
    cid = lax.axis_index("c")
    sid = lax.axis_index("s")
    start = lax.select(cid == 0, sid * C0, NS * C0 + sid * C1)
    n_groups = lax.select(cid == 0, C0 // GROUP, C1 // GROUP)
    sems = (sem0, sem1)

    def _issue_idx(g, slot_v):
        pltpu.async_copy(src_hbm.at[pl.ds(start + g * GROUP, GROUP)],
                         src_v.at[slot_v], isem)
        pltpu.async_copy(dst_hbm.at[pl.ds(start + g * GROUP, GROUP)],
                         dst_v.at[slot_v], isem)

    def _wait_idx(g, slot_v):
        pltpu.make_async_copy(src_hbm.at[pl.ds(start + g * GROUP, GROUP)],
                              src_v.at[slot_v], isem).wait()
        pltpu.make_async_copy(dst_hbm.at[pl.ds(start + g * GROUP, GROUP)],
                              dst_v.at[slot_v], isem).wait()

    def _zero_fill(i, carry):
        for j in range(DIM // 16):
            rows_v[0, i, pl.ds(j * 16, 16)] = jnp.zeros((16,), jnp.float32)
        return carry

    lax.fori_loop(0, CHUNK, _zero_fill, 0)
    for k in range(ROWS_PER_TILE // CHUNK):
        pltpu.sync_copy(rows_v.at[0],
                        acc_sh.at[pl.ds(sid * ROWS_PER_TILE + k * CHUNK, CHUNK)])

    plsc.subcore_barrier()

    pltpu.sync_copy(src_hbm.at[pl.ds(start, GROUP)], src_v.at[0])
    pltpu.sync_copy(dst_hbm.at[pl.ds(start, GROUP)], dst_v.at[0])
    pltpu.async_copy(h_hbm.at[src_v.at[0, 0]], rows_v.at[0], sem0)

    def _edge_group(g, carry):
        slot = lax.rem(g, 2)
        nslot = lax.rem(g + 1, 2)
        not_last = g + 1 < n_groups

        @pl.when(not_last)
        def _():
            _issue_idx(g + 1, nslot)

        for b in range(GROUP):
            cur = rows_v.at[b % 2]
            nxt = rows_v.at[(b + 1) % 2]
            if b + 1 < GROUP:
                pltpu.async_copy(h_hbm.at[src_v.at[slot, b + 1]], nxt,
                                 sems[(b + 1) % 2])
            else:
                @pl.when(not_last)
                def _():
                    _wait_idx(g + 1, nslot)
                    pltpu.async_copy(h_hbm.at[src_v.at[nslot, 0]], nxt,
                                     sems[(b + 1) % 2])
            pltpu.make_async_copy(h_hbm.at[src_v.at[slot, b]], cur,
                                  sems[b % 2]).wait()
            pltpu.sync_copy(cur, acc_sh.at[dst_v.at[slot, b]], add=True)
        return carry

    lax.fori_loop(0, n_groups, _edge_group, 0)

    plsc.subcore_barrier()

    base = sid * ROWS_PER_TILE
    pltpu.sync_copy(acc_sh.at[pl.ds(base, ROWS_PER_TILE)],
                    part_out.at[cid, pl.ds(base, ROWS_PER_TILE)])


def _make_sc_layer():
    mesh = plsc.VectorSubcoreMesh(core_axis_name="c", subcore_axis_name="s")
    return pl.kernel(
        _sc_layer_body,
        out_type=(jax.ShapeDtypeStruct((NC, ACC_ROWS, DIM), jnp.float32),),
        mesh=mesh,
        scratch_types=(
            pltpu.VMEM((2, GROUP, CHUNK), jnp.int32),
            pltpu.VMEM((2, GROUP, CHUNK), jnp.int32),
            pltpu.VMEM((2, CHUNK, DIM), jnp.float32),
            pltpu.VMEM_SHARED((ACC_ROWS, DIM), jnp.float32),
            pltpu.SemaphoreType.DMA,
            pltpu.SemaphoreType.DMA,
            pltpu.SemaphoreType.DMA,
        ),
    )


def _sc_counts_body(dst_hbm, cnt_out, dst_v, rows_v, cnt_sh):
    cid = lax.axis_index("c")
    sid = lax.axis_index("s")
    wid = cid * NS + sid
    per_worker = dst_hbm.shape[0] // NW
    start = wid * per_worker
    n_groups = per_worker // GROUP

    def _fill(val):
        def _body(i, carry):
            for j in range(DIM // 16):
                rows_v[i, pl.ds(j * 16, 16)] = jnp.full((16,), val, jnp.float32)
            return carry
        lax.fori_loop(0, CHUNK, _body, 0)

    _fill(0.0)
    for k in range(ROWS_PER_TILE // CHUNK):
        pltpu.sync_copy(rows_v, cnt_sh.at[pl.ds(sid * ROWS_PER_TILE + k * CHUNK, CHUNK)])
    _fill(1.0)

    plsc.subcore_barrier()

    def _edge_group(g, carry):
        pltpu.sync_copy(dst_hbm.at[pl.ds(start + g * GROUP, GROUP)], dst_v)
        for b in range(GROUP):
            pltpu.sync_copy(rows_v, cnt_sh.at[dst_v.at[b]], add=True)
        return carry

    lax.fori_loop(0, n_groups, _edge_group, 0)

    plsc.subcore_barrier()

    base = sid * ROWS_PER_TILE
    pltpu.sync_copy(cnt_sh.at[pl.ds(base, ROWS_PER_TILE)],
                    cnt_out.at[cid, pl.ds(base, ROWS_PER_TILE)])


def _make_sc_counts():
    mesh = plsc.VectorSubcoreMesh(core_axis_name="c", subcore_axis_name="s")
    return pl.kernel(
        _sc_counts_body,
        out_type=(jax.ShapeDtypeStruct((NC, ACC_ROWS, CNT_W), jnp.float32),),
        mesh=mesh,
        scratch_types=(
            pltpu.VMEM((GROUP, CHUNK), jnp.int32),
            pltpu.VMEM((CHUNK, CNT_W), jnp.float32),
            pltpu.VMEM_SHARED((ACC_ROWS, CNT_W), jnp.float32),
        ),
    )


def _tc_combine_body(first, part_ref, cnt_ref, fin_ref, h_ref, fout_ref):
    p = part_ref[0] + part_ref[1]
    c = cnt_ref[0, :, 0:1] + cnt_ref[1, :, 0:1]
    h = p / jnp.maximum(c, 1.0)
    h_ref[...] = h
    if first:
        fout_ref[...] = (fin_ref[...] + h) * 0.25
    else:
        fout_ref[...] = fin_ref[...] + h * 0.25


def _make_tc_combine(first):
    BR = 1000
    grid = (N_NODES // BR,)
    return pl.pallas_call(
        functools.partial(_tc_combine_body, first),
        grid=grid,
        in_specs=[
            pl.BlockSpec((NC, BR, DIM), lambda i: (0, i, 0)),
            pl.BlockSpec((NC, BR, CNT_W), lambda i: (0, i, 0)),
            pl.BlockSpec((BR, DIM), lambda i: (i, 0)),
        ],
        out_specs=[
            pl.BlockSpec((BR, DIM), lambda i: (i, 0)),
            pl.BlockSpec((BR, DIM), lambda i: (i, 0)),
        ],
        out_shape=[
            jax.ShapeDtypeStruct((N_NODES, DIM), jnp.float32),
            jax.ShapeDtypeStruct((N_NODES, DIM), jnp.float32),
        ],
    )


def kernel(x, edge_index):
    n_edges = edge_index.shape[1]
    pad_e = TOT_CHUNKS * CHUNK
    n_pad = pad_e - n_edges

    src = edge_index[0].astype(jnp.int32)
    dst = edge_index[1].astype(jnp.int32)
    src = jnp.concatenate(
        [src, jnp.zeros((n_pad,), jnp.int32)]).reshape(TOT_CHUNKS, CHUNK)
    pad_dst = DUMMY_ROW + (jnp.arange(n_pad, dtype=jnp.int32) % (ACC_ROWS - N_NODES))
    dst = jnp.concatenate([dst, pad_dst]).reshape(TOT_CHUNKS, CHUNK)

    sc_layer = _make_sc_layer()
    sc_counts = _make_sc_counts()
    tc_first = _make_tc_combine(first=True)
    tc_rest = _make_tc_combine(first=False)

    (cnt,) = sc_counts(dst)
    (part,) = sc_layer(x, src, dst)
    h, fin = tc_first(part, cnt, x)
    for _ in range(N_LAYERS - 1):
        (part,) = sc_layer(h, src, dst)
        h, fin = tc_rest(part, cnt, fin)
    return fin
